# baseline (speedup 1.0000x reference)
import jax
import jax.numpy as jnp
from jax.experimental import pallas as pl

_B, _C, _S = 256, 768, 196
_NC = 10
_BB = 8


def _body(f_ref, w_ref, b_ref, o_ref):
    f = f_ref[...]
    pooled = jnp.sum(f, axis=-1) * (1.0 / _S)
    acc = jax.lax.dot_general(
        pooled, w_ref[...], (((1,), (1,)), ((), ())),
        preferred_element_type=jnp.float32)
    o_ref[...] = acc + b_ref[...]


def kernel(features, W, b):
    f = features.reshape(_B, _C, _S)
    out = pl.pallas_call(
        _body,
        grid=(_B // _BB,),
        in_specs=[
            pl.BlockSpec((_BB, _C, _S), lambda i: (i, 0, 0)),
            pl.BlockSpec((_NC, _C), lambda i: (0, 0)),
            pl.BlockSpec((1, _NC), lambda i: (0, 0)),
        ],
        out_specs=pl.BlockSpec((_BB, _NC), lambda i: (i, 0)),
        out_shape=jax.ShapeDtypeStruct((_B, _NC), jnp.float32),
    )(f, W, b.reshape(1, _NC))
    return out
